# baseline (speedup 1.0000x reference)
import functools

import jax
import jax.numpy as jnp
from jax import lax
from jax.experimental import pallas as pl
from jax.experimental.pallas import tpu as pltpu
from jax.experimental.pallas import tpu_sc as plsc

_B = 8192
_G = 64
_D = 80
_NW = 32
_RPW = _B // _NW
_CR = 8
_NCH = _RPW // _CR
_ND16 = _D // 16

_CLIP_HI = _G - 1.0 - 1e-06


def _sc_body(spec_hbm, out_hbm, spec_v, out_v, il_v, sem0, sem1):
    wid = lax.axis_index("s") * 2 + lax.axis_index("c")
    base = wid * _RPW

    pltpu.sync_copy(spec_hbm.at[pl.ds(base, _RPW)], spec_v)

    zf = jnp.zeros((16,), jnp.float32)

    def zero_row(i, carry):
        slot = i // _CR
        r = lax.rem(i, _CR)
        for g in range(_G):
            for d16 in range(_ND16):
                out_v[slot, r, g, pl.ds(d16 * 16, 16)] = zf
        return carry

    lax.fori_loop(0, 2 * _CR, zero_row, 0)

    sems = (sem0, sem1)

    def copy_out(slot, c):
        return pltpu.make_async_copy(
            out_v.at[slot],
            out_hbm.at[pl.ds(base + c * _CR, _CR)],
            sems[slot],
        )

    def fill_chunk(slot, c, unscatter):
        for r in range(_CR):
            row = c * _CR + r
            for d16 in range(_ND16):
                d_idx = lax.iota(jnp.int32, 16) + (d16 * 16)
                tile = out_v.at[slot, r]
                if unscatter:
                    old = il_v[slot, r, pl.ds(d16 * 16, 16)]
                    plsc.store_scatter(tile, [old, d_idx], zf)
                    plsc.store_scatter(tile, [old + 1, d_idx], zf)
                s = spec_v[row, pl.ds(d16 * 16, 16)]
                sc = jnp.minimum(jnp.maximum(s, 0.0), _CLIP_HI)
                il = sc.astype(jnp.int32)
                frac = sc - il.astype(jnp.float32)
                plsc.store_scatter(tile, [il, d_idx], 1.0 - frac)
                plsc.store_scatter(tile, [il + 1, d_idx], frac)
                il_v[slot, r, pl.ds(d16 * 16, 16)] = il

    fill_chunk(0, 0, False)
    copy_out(0, 0).start()
    fill_chunk(1, 1, False)
    copy_out(1, 1).start()

    def step(c2, carry):
        c0 = 2 * c2
        copy_out(0, c0 - 2).wait()
        fill_chunk(0, c0, True)
        copy_out(0, c0).start()
        copy_out(1, c0 - 1).wait()
        fill_chunk(1, c0 + 1, True)
        copy_out(1, c0 + 1).start()
        return carry

    lax.fori_loop(1, _NCH // 2, step, 0)

    copy_out(0, _NCH - 2).wait()
    copy_out(1, _NCH - 1).wait()


def kernel(spec):
    k = functools.partial(
        pl.kernel,
        mesh=plsc.VectorSubcoreMesh(core_axis_name="c", subcore_axis_name="s"),
        out_type=jax.ShapeDtypeStruct((_B, _G, _D), jnp.float32),
        scratch_types=[
            pltpu.VMEM((_RPW, _D), jnp.float32),
            pltpu.VMEM((2, _CR, _G, _D), jnp.float32),
            pltpu.VMEM((2, _CR, _D), jnp.int32),
            pltpu.SemaphoreType.DMA,
            pltpu.SemaphoreType.DMA,
        ],
        compiler_params=pltpu.CompilerParams(
            needs_layout_passes=False, use_tc_tiling_on_sc=False
        ),
    )(_sc_body)
    return k(spec)


# speedup vs baseline: 1.5808x; 1.5808x over previous
import jax
import jax.numpy as jnp
from jax.experimental import pallas as pl
from jax.experimental.pallas import tpu as pltpu

_G = 64
_BB = 128
_NBUF = 8


def _twohot_body(spec_ref, out_ref, scratch, sems):
    b = out_ref.shape[0]
    d = out_ref.shape[2]
    nchunk = b // _BB

    def chunk_copy(c, slot):
        return pltpu.make_async_copy(
            scratch.at[pl.ds(slot * _BB, _BB)],
            out_ref.at[pl.ds(c * _BB, _BB)],
            sems.at[slot],
        )

    def step(c, carry):
        slot = jax.lax.rem(c, _NBUF)

        @pl.when(c >= _NBUF)
        def _():
            chunk_copy(c - _NBUF, slot).wait()

        s = spec_ref[pl.ds(c * _BB, _BB), :]
        sc = jnp.clip(s, 0.0, _G - 1.0 - 1e-06)
        lower = jnp.floor(sc)
        frac = sc - lower
        il = lower.astype(jnp.int32)[:, None, :]
        f = frac[:, None, :]
        g = jax.lax.broadcasted_iota(jnp.int32, (_BB, _G, d), 1)
        scratch[pl.ds(slot * _BB, _BB)] = jnp.where(
            g == il, 1.0 - f, jnp.where(g == il + 1, f, 0.0)
        )
        chunk_copy(c, slot).start()
        return carry

    jax.lax.fori_loop(0, nchunk, step, 0)

    def drain(i, carry):
        c = nchunk - _NBUF + i
        chunk_copy(c, jax.lax.rem(c, _NBUF)).wait()
        return carry

    jax.lax.fori_loop(0, _NBUF, drain, 0)


def kernel(spec):
    b, d = spec.shape
    return pl.pallas_call(
        _twohot_body,
        in_specs=[pl.BlockSpec(memory_space=pltpu.MemorySpace.VMEM)],
        out_specs=pl.BlockSpec(memory_space=pl.ANY),
        out_shape=jax.ShapeDtypeStruct((b, _G, d), jnp.float32),
        scratch_shapes=[
            pltpu.VMEM((_NBUF * _BB, _G, d), jnp.float32),
            pltpu.SemaphoreType.DMA((_NBUF,)),
        ],
    )(spec)
